# baseline (speedup 1.0000x reference)
import jax
import jax.numpy as jnp
from jax.experimental import pallas as pl
from jax.experimental.pallas import tpu as pltpu

_BI = 200


def _layer_body(a_ref, tj_ref, ti_ref, w_ref, b_ref, eps_ref, o_ref):
    a = a_ref[...].astype(jnp.bfloat16)
    t = tj_ref[...].astype(jnp.bfloat16)
    agg = jnp.dot(a, t, preferred_element_type=jnp.float32)
    pre = (1.0 + eps_ref[0]) * ti_ref[...] + agg
    y = jnp.dot(pre.astype(jnp.bfloat16), w_ref[...].astype(jnp.bfloat16),
                preferred_element_type=jnp.float32) + b_ref[...]
    o_ref[...] = jnp.maximum(y, 0.0)


def _layer(A, t_in, W, b, eps_i):
    n = A.shape[0]
    k = t_in.shape[1]
    m = W.shape[1]
    ni = n // _BI
    return pl.pallas_call(
        _layer_body,
        grid=(ni,),
        in_specs=[
            pl.BlockSpec((_BI, n), lambda i: (i, 0)),
            pl.BlockSpec((n, k), lambda i: (0, 0)),
            pl.BlockSpec((_BI, k), lambda i: (i, 0)),
            pl.BlockSpec((k, m), lambda i: (0, 0)),
            pl.BlockSpec((1, m), lambda i: (0, 0)),
            pl.BlockSpec(memory_space=pltpu.SMEM),
        ],
        out_specs=pl.BlockSpec((_BI, m), lambda i: (i, 0)),
        out_shape=jax.ShapeDtypeStruct((n, m), jnp.float32),
        compiler_params=pltpu.CompilerParams(
            dimension_semantics=("arbitrary",)),
    )(A, t_in, t_in, W, b.reshape(1, m), eps_i.reshape(1,))


def kernel(A, X, epsilon_dim, h, W0, b0, W1, b1, W2, b2, eps):
    n = X.shape[0]
    eps_dim = W0.shape[0] - X.shape[1] - h.shape[1]
    bern = jax.random.bernoulli(jax.random.key(42), 0.5, (n, eps_dim)).astype(jnp.float32)
    t = jnp.concatenate([X, bern, h], axis=1)
    for i, (W, b) in enumerate(((W0, b0), (W1, b1), (W2, b2))):
        t = _layer(A, t, W, b, eps[i])
    return t


# speedup vs baseline: 1.0037x; 1.0037x over previous
import jax
import jax.numpy as jnp
from jax.experimental import pallas as pl
from jax.experimental.pallas import tpu as pltpu

_BI = 200


def _layer_body(a_ref, tj_ref, ti_ref, w_ref, b_ref, eps_ref, o_ref):
    agg = jnp.dot(a_ref[...], tj_ref[...], preferred_element_type=jnp.float32,
                  precision=jax.lax.Precision.DEFAULT)
    pre = (1.0 + eps_ref[0]) * ti_ref[...] + agg
    y = jnp.dot(pre, w_ref[...], preferred_element_type=jnp.float32,
                precision=jax.lax.Precision.DEFAULT) + b_ref[...]
    o_ref[...] = jnp.maximum(y, 0.0)


def _layer(A, t_in, W, b, eps_i):
    n = A.shape[0]
    k = t_in.shape[1]
    m = W.shape[1]
    ni = n // _BI
    return pl.pallas_call(
        _layer_body,
        grid=(ni,),
        in_specs=[
            pl.BlockSpec((_BI, n), lambda i: (i, 0)),
            pl.BlockSpec((n, k), lambda i: (0, 0)),
            pl.BlockSpec((_BI, k), lambda i: (i, 0)),
            pl.BlockSpec((k, m), lambda i: (0, 0)),
            pl.BlockSpec((1, m), lambda i: (0, 0)),
            pl.BlockSpec(memory_space=pltpu.SMEM),
        ],
        out_specs=pl.BlockSpec((_BI, m), lambda i: (i, 0)),
        out_shape=jax.ShapeDtypeStruct((n, m), jnp.float32),
        compiler_params=pltpu.CompilerParams(
            dimension_semantics=("arbitrary",)),
    )(A, t_in, t_in, W, b.reshape(1, m), eps_i.reshape(1,))


def kernel(A, X, epsilon_dim, h, W0, b0, W1, b1, W2, b2, eps):
    n = X.shape[0]
    eps_dim = W0.shape[0] - X.shape[1] - h.shape[1]
    bern = jax.random.bernoulli(jax.random.key(42), 0.5, (n, eps_dim)).astype(jnp.float32)
    t = jnp.concatenate([X, bern, h], axis=1)
    for i, (W, b) in enumerate(((W0, b0), (W1, b1), (W2, b2))):
        t = _layer(A, t, W, b, eps[i])
    return t
